# baseline (speedup 1.0000x reference)
import functools

import jax
import jax.numpy as jnp
from jax import lax
from jax.experimental import pallas as pl
from jax.experimental.pallas import tpu as pltpu
from jax.experimental.pallas import tpu_sc as plsc

N = 10000
E = 320000
D = 128
H = 32

NC = 2
NS = 16
NW = NC * NS
K = 128
C = 80
E_PAD = NW * C * K
TOT = E_PAD // K
CT = TOT // NS
N_PAD = 10240
RPT = N_PAD // NS

_mesh = plsc.VectorSubcoreMesh(core_axis_name="c", subcore_axis_name="s")


@functools.partial(
    pl.kernel,
    out_type=jax.ShapeDtypeStruct((NC, N_PAD), jnp.float32),
    mesh=_mesh,
    compiler_params=pltpu.CompilerParams(use_tc_tiling_on_sc=False),
    scratch_types=[
        pltpu.VMEM((C, K), jnp.int32),
        pltpu.VMEM((K,), jnp.float32),
        pltpu.VMEM_SHARED((N_PAD,), jnp.float32),
    ],
)
def _sc_degree(dst_hbm, zeros_hbm, ones_hbm, out_hbm, dst_v, ones_v, acc):
    c = lax.axis_index("c")
    s = lax.axis_index("s")
    wid = c * NS + s
    pltpu.sync_copy(dst_hbm.at[pl.ds(wid * C, C)], dst_v)
    pltpu.sync_copy(ones_hbm, ones_v)
    pltpu.sync_copy(zeros_hbm.at[pl.ds(s * RPT, RPT)],
                    acc.at[pl.ds(s * RPT, RPT)])
    plsc.subcore_barrier()

    def body(j, carry):
        pltpu.sync_copy(ones_v, acc.at[dst_v.at[j]], add=True)
        return carry

    lax.fori_loop(0, C, body, 0)
    plsc.subcore_barrier()
    pltpu.sync_copy(acc.at[pl.ds(s * RPT, RPT)],
                    out_hbm.at[c, pl.ds(s * RPT, RPT)])


@functools.partial(
    pl.kernel,
    out_type=jax.ShapeDtypeStruct((N_PAD, H), jnp.float32),
    mesh=_mesh,
    compiler_params=pltpu.CompilerParams(use_tc_tiling_on_sc=False),
    scratch_types=[
        pltpu.VMEM((CT, K), jnp.int32),
        pltpu.VMEM((CT, K), jnp.int32),
        [pltpu.VMEM((K, H), jnp.float32)] * 4,
        [pltpu.SemaphoreType.DMA] * 4,
        pltpu.VMEM_SHARED((N_PAD, H), jnp.float32),
    ],
)
def _sc_aggregate(hs_hbm, src_hbm, dst_hbm, zeros_hbm,
                  out_hbm, src_v, dst_v, rows, gsems, acc):
    c = lax.axis_index("c")
    s = lax.axis_index("s")
    NB = 4

    @pl.when(c == 0)
    def _():
        pltpu.sync_copy(src_hbm.at[pl.ds(s * CT, CT)], src_v)
        pltpu.sync_copy(dst_hbm.at[pl.ds(s * CT, CT)], dst_v)
        pltpu.sync_copy(zeros_hbm.at[pl.ds(s * RPT, RPT)],
                        acc.at[pl.ds(s * RPT, RPT)])
        plsc.subcore_barrier()

        for t in range(NB):
            pltpu.async_copy(hs_hbm.at[src_v.at[t]], rows[t], gsems[t])

        def stage(j, t, prefetch):
            pltpu.make_async_copy(hs_hbm.at[src_v.at[j]], rows[t],
                                  gsems[t]).wait()
            pltpu.sync_copy(rows[t], acc.at[dst_v.at[j]], add=True)
            if prefetch:
                pltpu.async_copy(hs_hbm.at[src_v.at[j + NB]], rows[t],
                                 gsems[t])

        def body(i, carry):
            j = i * NB
            for t in range(NB):
                stage(j + t, t, True)
            return carry

        lax.fori_loop(0, CT // NB - 1, body, 0)
        for t in range(NB):
            stage(CT - NB + t, t, False)

        plsc.subcore_barrier()
        pltpu.sync_copy(acc.at[pl.ds(s * RPT, RPT)],
                        out_hbm.at[pl.ds(s * RPT, RPT)])


BR = 1024


def _tc1_body(x_ref, w1_ref, degp_ref, hs_ref):
    deg = degp_ref[:, 0] + degp_ref[:, 1] + 1.0
    dinv = lax.rsqrt(deg)
    h = jnp.dot(x_ref[...], w1_ref[...], preferred_element_type=jnp.float32)
    hs_ref[...] = h * dinv[:, None]


def _tc2_body(aggp_ref, hs1_ref, degp_ref, b1_ref, w2_ref, r_ref, hs2_ref):
    deg = degp_ref[:, 0] + degp_ref[:, 1] + 1.0
    dinv = lax.rsqrt(deg)
    ssum = aggp_ref[...] + hs1_ref[...]
    agg1 = ssum * dinv[:, None] + b1_ref[...]
    r = jnp.maximum(agg1, 0.0)
    r_ref[...] = r
    h2 = jnp.dot(r, w2_ref[...], preferred_element_type=jnp.float32)
    hs2_ref[...] = h2 * dinv[:, None]


def _tc3_body(aggp_ref, hs2_ref, r_ref, degp_ref, b2_ref, wout_ref,
              bout_ref, out_ref):
    deg = degp_ref[:, 0] + degp_ref[:, 1] + 1.0
    dinv = lax.rsqrt(deg)
    ssum = aggp_ref[...] + hs2_ref[...]
    agg2 = ssum * dinv[:, None] + b2_ref[...]
    hout = r_ref[...] + agg2
    z = jnp.dot(hout, wout_ref[...], preferred_element_type=jnp.float32)
    out_ref[...] = jnp.tanh(z + bout_ref[...]) * 5.0


def _row_spec(h):
    return pl.BlockSpec((BR, h), lambda i: (i, 0))


def _bcast_spec(shape):
    nd = len(shape)
    return pl.BlockSpec(shape, lambda i: (0,) * nd)


_degp_spec = pl.BlockSpec((BR, 2), lambda i: (i, 0))
_aggp_spec = _row_spec(H)
_grid = (N_PAD // BR,)


BR1 = 1000


def _tc1(x, w1, degp):
    return pl.pallas_call(
        _tc1_body,
        grid=(N // BR1,),
        in_specs=[pl.BlockSpec((BR1, D), lambda i: (i, 0)),
                  _bcast_spec((D, H)),
                  pl.BlockSpec((BR1, 2), lambda i: (i, 0))],
        out_specs=pl.BlockSpec((BR1, H), lambda i: (i, 0)),
        out_shape=jax.ShapeDtypeStruct((N_PAD, H), jnp.float32),
    )(x, w1, degp)


def _tc2(aggp, hs1, degp, b1, w2):
    return pl.pallas_call(
        _tc2_body,
        grid=_grid,
        in_specs=[_aggp_spec, _row_spec(H), _degp_spec,
                  _bcast_spec((1, H)), _bcast_spec((H, H))],
        out_specs=[_row_spec(H), _row_spec(H)],
        out_shape=[jax.ShapeDtypeStruct((N_PAD, H), jnp.float32),
                   jax.ShapeDtypeStruct((N_PAD, H), jnp.float32)],
    )(aggp, hs1, degp, b1, w2)


def _tc3(aggp, hs2, r, degp, b2, w_out, b_out):
    return pl.pallas_call(
        _tc3_body,
        grid=_grid,
        in_specs=[_aggp_spec, _row_spec(H), _row_spec(H), _degp_spec,
                  _bcast_spec((1, H)), _bcast_spec((H, 1)),
                  _bcast_spec((1, 1))],
        out_specs=_row_spec(1),
        out_shape=jax.ShapeDtypeStruct((N_PAD, 1), jnp.float32),
    )(aggp, hs2, r, degp, b2, w_out, b_out)


def kernel(x, edge_index, W1, b1, W2, b2, W_out, b_out):
    src = edge_index[0]
    dst = edge_index[1]
    pad = jnp.full((E_PAD - E,), N, dtype=jnp.int32)
    srcf = jnp.concatenate([src, pad]).reshape(TOT, K)
    dstf = jnp.concatenate([dst, pad]).reshape(TOT, K)

    zeros1 = jnp.zeros((N_PAD,), jnp.float32)
    zeros2 = jnp.zeros((N_PAD, H), jnp.float32)
    ones = jnp.ones((K,), jnp.float32)

    degp = _sc_degree(dstf, zeros1, ones).T
    hs1 = _tc1(x, W1, degp)
    aggp1 = _sc_aggregate(hs1, srcf, dstf, zeros2)
    r, hs2 = _tc2(aggp1, hs1, degp, b1.reshape(1, H), W2)
    aggp2 = _sc_aggregate(hs2, srcf, dstf, zeros2)
    out = _tc3(aggp2, hs2, r, degp, b2.reshape(1, H), W_out,
               b_out.reshape(1, 1))
    return out[:N]


# speedup vs baseline: 1.1959x; 1.1959x over previous
import functools

import jax
import jax.numpy as jnp
from jax import lax
from jax.experimental import pallas as pl
from jax.experimental.pallas import tpu as pltpu
from jax.experimental.pallas import tpu_sc as plsc

N = 10000
E = 320000
D = 128
H = 32

NC = 2
NS = 16
NW = NC * NS
K = 128
C = 80
E_PAD = NW * C * K
TOT = E_PAD // K
CN = 120
CS = (TOT // NS) - CN
N_PAD = 10240
RPT = N_PAD // NS

_mesh = plsc.VectorSubcoreMesh(core_axis_name="c", subcore_axis_name="s")


@functools.partial(
    pl.kernel,
    out_type=jax.ShapeDtypeStruct((NC, N_PAD), jnp.float32),
    mesh=_mesh,
    scratch_types=[
        pltpu.VMEM((C, K), jnp.int32),
        pltpu.VMEM((K,), jnp.float32),
        pltpu.VMEM_SHARED((N_PAD,), jnp.float32),
    ],
)
def _sc_degree(dst_hbm, zeros_hbm, ones_hbm, out_hbm, dst_v, ones_v, acc):
    c = lax.axis_index("c")
    s = lax.axis_index("s")
    wid = c * NS + s
    pltpu.sync_copy(dst_hbm.at[wid], dst_v)
    pltpu.sync_copy(ones_hbm, ones_v)
    pltpu.sync_copy(zeros_hbm.at[pl.ds(s * RPT, RPT)],
                    acc.at[pl.ds(s * RPT, RPT)])
    plsc.subcore_barrier()

    def body(j, carry):
        pltpu.sync_copy(ones_v, acc.at[dst_v.at[j]], add=True)
        return carry

    lax.fori_loop(0, C, body, 0)
    plsc.subcore_barrier()
    pltpu.sync_copy(acc.at[pl.ds(s * RPT, RPT)],
                    out_hbm.at[c, pl.ds(s * RPT, RPT)])


@functools.partial(
    pl.kernel,
    out_type=jax.ShapeDtypeStruct((NC, N_PAD, H), jnp.float32),
    mesh=_mesh,
    compiler_params=pltpu.CompilerParams(use_tc_tiling_on_sc=False),
    scratch_types=[
        pltpu.VMEM((CN, K), jnp.int32),
        pltpu.VMEM((CN, K), jnp.int32),
        [pltpu.VMEM((K, H), jnp.float32)] * 4,
        [pltpu.SemaphoreType.DMA] * 4,
        pltpu.VMEM_SHARED((N_PAD, H), jnp.float32),
    ],
)
def _sc_aggregate(hs_hbm, srcn_hbm, dstn_hbm, srcs_hbm, dsts_hbm, zeros_hbm,
                  out_hbm, src_v, dst_v, rows, gsems, acc):
    c = lax.axis_index("c")
    s = lax.axis_index("s")
    pltpu.sync_copy(zeros_hbm.at[pl.ds(s * RPT, RPT)],
                    acc.at[pl.ds(s * RPT, RPT)])

    NB = 4

    def run(cnt):
        for t in range(NB):
            pltpu.async_copy(hs_hbm.at[src_v.at[t]], rows[t], gsems[t])

        def stage(j, t, prefetch):
            pltpu.make_async_copy(hs_hbm.at[src_v.at[j]], rows[t],
                                  gsems[t]).wait()
            pltpu.sync_copy(rows[t], acc.at[dst_v.at[j]], add=True)
            if prefetch:
                pltpu.async_copy(hs_hbm.at[src_v.at[j + NB]], rows[t],
                                 gsems[t])

        def body(i, carry):
            j = i * NB
            for t in range(NB):
                stage(j + t, t, True)
            return carry

        lax.fori_loop(0, cnt // NB - 1, body, 0)
        for t in range(NB):
            stage(cnt - NB + t, t, False)

    @pl.when(c == 0)
    def _():
        pltpu.sync_copy(srcn_hbm.at[s], src_v)
        pltpu.sync_copy(dstn_hbm.at[s], dst_v)
        plsc.subcore_barrier()
        run(CN)

    @pl.when(c == 1)
    def _():
        pltpu.sync_copy(srcs_hbm.at[s], src_v.at[pl.ds(0, CS)])
        pltpu.sync_copy(dsts_hbm.at[s], dst_v.at[pl.ds(0, CS)])
        plsc.subcore_barrier()
        run(CS)

    plsc.subcore_barrier()
    pltpu.sync_copy(acc.at[pl.ds(s * RPT, RPT)],
                    out_hbm.at[c, pl.ds(s * RPT, RPT)])


BR = 1024


def _tc1_body(x_ref, w1_ref, degp_ref, hs_ref):
    deg = degp_ref[:, 0] + degp_ref[:, 1] + 1.0
    dinv = lax.rsqrt(deg)
    h = jnp.dot(x_ref[...], w1_ref[...], preferred_element_type=jnp.float32)
    hs_ref[...] = h * dinv[:, None]


def _tc2_body(aggp_ref, hs1_ref, degp_ref, b1_ref, w2_ref, r_ref, hs2_ref):
    deg = degp_ref[:, 0] + degp_ref[:, 1] + 1.0
    dinv = lax.rsqrt(deg)
    ssum = aggp_ref[0] + aggp_ref[1] + hs1_ref[...]
    agg1 = ssum * dinv[:, None] + b1_ref[...]
    r = jnp.maximum(agg1, 0.0)
    r_ref[...] = r
    h2 = jnp.dot(r, w2_ref[...], preferred_element_type=jnp.float32)
    hs2_ref[...] = h2 * dinv[:, None]


def _tc3_body(aggp_ref, hs2_ref, r_ref, degp_ref, b2_ref, wout_ref,
              bout_ref, out_ref):
    deg = degp_ref[:, 0] + degp_ref[:, 1] + 1.0
    dinv = lax.rsqrt(deg)
    ssum = aggp_ref[0] + aggp_ref[1] + hs2_ref[...]
    agg2 = ssum * dinv[:, None] + b2_ref[...]
    hout = r_ref[...] + agg2
    z = jnp.dot(hout, wout_ref[...], preferred_element_type=jnp.float32)
    out_ref[...] = jnp.tanh(z + bout_ref[...]) * 5.0


def _row_spec(h):
    return pl.BlockSpec((BR, h), lambda i: (i, 0))


def _bcast_spec(shape):
    nd = len(shape)
    return pl.BlockSpec(shape, lambda i: (0,) * nd)


_degp_spec = pl.BlockSpec((BR, 2), lambda i: (i, 0))
_aggp_spec = pl.BlockSpec((2, BR, H), lambda i: (0, i, 0))
_grid = (N_PAD // BR,)


BR1 = 1000


def _tc1(x, w1, degp):
    return pl.pallas_call(
        _tc1_body,
        grid=(N // BR1,),
        in_specs=[pl.BlockSpec((BR1, D), lambda i: (i, 0)),
                  _bcast_spec((D, H)),
                  pl.BlockSpec((BR1, 2), lambda i: (i, 0))],
        out_specs=pl.BlockSpec((BR1, H), lambda i: (i, 0)),
        out_shape=jax.ShapeDtypeStruct((N_PAD, H), jnp.float32),
    )(x, w1, degp)


def _tc2(aggp, hs1, degp, b1, w2):
    return pl.pallas_call(
        _tc2_body,
        grid=_grid,
        in_specs=[_aggp_spec, _row_spec(H), _degp_spec,
                  _bcast_spec((1, H)), _bcast_spec((H, H))],
        out_specs=[_row_spec(H), _row_spec(H)],
        out_shape=[jax.ShapeDtypeStruct((N_PAD, H), jnp.float32),
                   jax.ShapeDtypeStruct((N_PAD, H), jnp.float32)],
    )(aggp, hs1, degp, b1, w2)


def _tc3(aggp, hs2, r, degp, b2, w_out, b_out):
    return pl.pallas_call(
        _tc3_body,
        grid=_grid,
        in_specs=[_aggp_spec, _row_spec(H), _row_spec(H), _degp_spec,
                  _bcast_spec((1, H)), _bcast_spec((H, 1)),
                  _bcast_spec((1, 1))],
        out_specs=_row_spec(1),
        out_shape=jax.ShapeDtypeStruct((N_PAD, 1), jnp.float32),
    )(aggp, hs2, r, degp, b2, w_out, b_out)


def kernel(x, edge_index, W1, b1, W2, b2, W_out, b_out):
    src = edge_index[0]
    dst = edge_index[1]
    pad = jnp.full((E_PAD - E,), N, dtype=jnp.int32)
    srcf = jnp.concatenate([src, pad]).reshape(TOT, K)
    dstf = jnp.concatenate([dst, pad]).reshape(TOT, K)
    dst3 = dstf.reshape(NW, C, K)
    srcn = srcf[:NS * CN].reshape(NS, CN, K)
    srcs = srcf[NS * CN:].reshape(NS, CS, K)
    dstn = dstf[:NS * CN].reshape(NS, CN, K)
    dsts = dstf[NS * CN:].reshape(NS, CS, K)

    zeros1 = jnp.zeros((N_PAD,), jnp.float32)
    zeros2 = jnp.zeros((N_PAD, H), jnp.float32)
    ones = jnp.ones((K,), jnp.float32)

    degp = _sc_degree(dst3, zeros1, ones).T
    hs1 = _tc1(x, W1, degp)
    aggp1 = _sc_aggregate(hs1, srcn, dstn, srcs, dsts, zeros2)
    r, hs2 = _tc2(aggp1, hs1, degp, b1.reshape(1, H), W2)
    aggp2 = _sc_aggregate(hs2, srcn, dstn, srcs, dsts, zeros2)
    out = _tc3(aggp2, hs2, r, degp, b2.reshape(1, H), W_out,
               b_out.reshape(1, 1))
    return out[:N]


# speedup vs baseline: 1.2737x; 1.0650x over previous
import functools

import jax
import jax.numpy as jnp
from jax import lax
from jax.experimental import pallas as pl
from jax.experimental.pallas import tpu as pltpu
from jax.experimental.pallas import tpu_sc as plsc

N = 10000
E = 320000
D = 128
H = 32

NC = 2
NS = 16
NW = NC * NS
K = 128
C = 80
E_PAD = NW * C * K
TOT = E_PAD // K
CN = 120
CS = (TOT // NS) - CN
N_PAD = 10240
RPT = N_PAD // NS

_mesh = plsc.VectorSubcoreMesh(core_axis_name="c", subcore_axis_name="s")


@functools.partial(
    pl.kernel,
    out_type=jax.ShapeDtypeStruct((NC, N_PAD), jnp.float32),
    mesh=_mesh,
    scratch_types=[
        pltpu.VMEM((C, K), jnp.int32),
        pltpu.VMEM((K,), jnp.float32),
        pltpu.VMEM_SHARED((N_PAD,), jnp.float32),
    ],
)
def _sc_degree(dst_hbm, zeros_hbm, ones_hbm, out_hbm, dst_v, ones_v, acc):
    c = lax.axis_index("c")
    s = lax.axis_index("s")
    wid = c * NS + s
    pltpu.sync_copy(dst_hbm.at[wid], dst_v)
    pltpu.sync_copy(ones_hbm, ones_v)
    pltpu.sync_copy(zeros_hbm.at[pl.ds(s * RPT, RPT)],
                    acc.at[pl.ds(s * RPT, RPT)])
    plsc.subcore_barrier()

    def body(j, carry):
        pltpu.sync_copy(ones_v, acc.at[dst_v.at[j]], add=True)
        return carry

    lax.fori_loop(0, C, body, 0)
    plsc.subcore_barrier()
    pltpu.sync_copy(acc.at[pl.ds(s * RPT, RPT)],
                    out_hbm.at[c, pl.ds(s * RPT, RPT)])


@functools.partial(
    pl.kernel,
    out_type=jax.ShapeDtypeStruct((NC, N_PAD, H), jnp.float32),
    mesh=_mesh,
    compiler_params=pltpu.CompilerParams(use_tc_tiling_on_sc=False),
    scratch_types=[
        pltpu.VMEM((CN, K), jnp.int32),
        pltpu.VMEM((CN, K), jnp.int32),
        [pltpu.VMEM((K, H), jnp.float32)] * 4,
        [pltpu.SemaphoreType.DMA] * 4,
        pltpu.VMEM_SHARED((N_PAD, H), jnp.float32),
    ],
)
def _sc_aggregate(hs_hbm, srcn_hbm, dstn_hbm, srcs_hbm, dsts_hbm, zeros_hbm,
                  out_hbm, src_v, dst_v, rows, gsems, acc):
    c = lax.axis_index("c")
    s = lax.axis_index("s")
    pltpu.sync_copy(zeros_hbm.at[pl.ds(s * RPT, RPT)],
                    acc.at[pl.ds(s * RPT, RPT)])

    NB = 4

    def run(cnt):
        for t in range(NB):
            pltpu.async_copy(hs_hbm.at[src_v.at[t]], rows[t], gsems[t])

        def stage(j, t, prefetch):
            pltpu.make_async_copy(hs_hbm.at[src_v.at[j]], rows[t],
                                  gsems[t]).wait()
            pltpu.sync_copy(rows[t], acc.at[dst_v.at[j]], add=True)
            if prefetch:
                pltpu.async_copy(hs_hbm.at[src_v.at[j + NB]], rows[t],
                                 gsems[t])

        def body(i, carry):
            j = i * NB
            for t in range(NB):
                stage(j + t, t, True)
            return carry

        lax.fori_loop(0, cnt // NB - 1, body, 0)
        for t in range(NB):
            stage(cnt - NB + t, t, False)

    @pl.when(c == 0)
    def _():
        pltpu.sync_copy(srcn_hbm.at[s], src_v)
        pltpu.sync_copy(dstn_hbm.at[s], dst_v)
        plsc.subcore_barrier()
        run(CN)

    @pl.when(c == 1)
    def _():
        pltpu.sync_copy(srcs_hbm.at[s], src_v.at[pl.ds(0, CS)])
        pltpu.sync_copy(dsts_hbm.at[s], dst_v.at[pl.ds(0, CS)])
        plsc.subcore_barrier()
        run(CS)

    plsc.subcore_barrier()
    pltpu.sync_copy(acc.at[pl.ds(s * RPT, RPT)],
                    out_hbm.at[c, pl.ds(s * RPT, RPT)])


import numpy as np

NP4 = N_PAD // 4
BP = 512
HP = 4 * H

_EXPAND = np.kron(np.eye(4, dtype=np.float32),
                  np.ones((1, H), np.float32))


def _dinv128(degp_ref, exp_ref):
    deg = degp_ref[0] + degp_ref[1] + 1.0
    dinv4 = lax.rsqrt(deg)
    return jnp.dot(dinv4, exp_ref[...], preferred_element_type=jnp.float32)


def _tc1_body(x0_ref, x1_ref, x2_ref, x3_ref, w1_ref, degp_ref, exp_ref,
              hs_ref):
    w1 = w1_ref[...]
    hs = [jnp.dot(xr[...], w1, preferred_element_type=jnp.float32)
          for xr in (x0_ref, x1_ref, x2_ref, x3_ref)]
    hp = jnp.concatenate(hs, axis=1)
    hs_ref[...] = hp * _dinv128(degp_ref, exp_ref)


def _tc2_body(aggp_ref, hs1_ref, degp_ref, exp_ref, b1_ref, w2_ref,
              r_ref, hs2_ref):
    dinv = _dinv128(degp_ref, exp_ref)
    ssum = aggp_ref[0] + aggp_ref[1] + hs1_ref[...]
    agg1 = ssum * dinv + b1_ref[...]
    r = jnp.maximum(agg1, 0.0)
    r_ref[...] = r
    h2 = jnp.dot(r, w2_ref[...], preferred_element_type=jnp.float32)
    hs2_ref[...] = h2 * dinv


def _tc3_body(aggp_ref, hs2_ref, r_ref, degp_ref, exp_ref, b2_ref, wout_ref,
              bout_ref, out_ref):
    dinv = _dinv128(degp_ref, exp_ref)
    ssum = aggp_ref[0] + aggp_ref[1] + hs2_ref[...]
    agg2 = ssum * dinv + b2_ref[...]
    hout = r_ref[...] + agg2
    z = jnp.dot(hout, wout_ref[...], preferred_element_type=jnp.float32)
    out_ref[...] = jnp.tanh(z + bout_ref[...]) * 5.0


def _prow_spec(h):
    return pl.BlockSpec((BP, h), lambda i: (i, 0))


def _bcast_spec(shape):
    nd = len(shape)
    return pl.BlockSpec(shape, lambda i: (0,) * nd)


_degp_spec = pl.BlockSpec((2, BP, 4), lambda i: (0, i, 0))
_aggp_spec = pl.BlockSpec((2, BP, HP), lambda i: (0, i, 0))
_grid = (NP4 // BP,)


_NBLK = NP4 // BP


def _tc1(x_pad, w1, degp4):
    xspecs = [pl.BlockSpec((BP, D), functools.partial(
        lambda i, k: (i + _NBLK * k, 0), k=k)) for k in range(4)]
    return pl.pallas_call(
        _tc1_body,
        grid=_grid,
        in_specs=xspecs + [_bcast_spec((D, H)), _degp_spec,
                           _bcast_spec((4, HP))],
        out_specs=_prow_spec(HP),
        out_shape=jax.ShapeDtypeStruct((NP4, HP), jnp.float32),
    )(x_pad, x_pad, x_pad, x_pad, w1, degp4, jnp.asarray(_EXPAND))


def _tc2(aggp, hs1, degp4, b1p, w2d):
    return pl.pallas_call(
        _tc2_body,
        grid=_grid,
        in_specs=[_aggp_spec, _prow_spec(HP), _degp_spec,
                  _bcast_spec((4, HP)), _bcast_spec((1, HP)),
                  _bcast_spec((HP, HP))],
        out_specs=[_prow_spec(HP), _prow_spec(HP)],
        out_shape=[jax.ShapeDtypeStruct((NP4, HP), jnp.float32),
                   jax.ShapeDtypeStruct((NP4, HP), jnp.float32)],
    )(aggp, hs1, degp4, jnp.asarray(_EXPAND), b1p, w2d)


def _tc3(aggp, hs2, r, degp4, b2p, woutd, boutp):
    return pl.pallas_call(
        _tc3_body,
        grid=_grid,
        in_specs=[_aggp_spec, _prow_spec(HP), _prow_spec(HP), _degp_spec,
                  _bcast_spec((4, HP)), _bcast_spec((1, HP)),
                  _bcast_spec((HP, 4)), _bcast_spec((1, 4))],
        out_specs=_prow_spec(4),
        out_shape=jax.ShapeDtypeStruct((NP4, 4), jnp.float32),
    )(aggp, hs2, r, degp4, jnp.asarray(_EXPAND), b2p, woutd, boutp)


def kernel(x, edge_index, W1, b1, W2, b2, W_out, b_out):
    src = edge_index[0]
    dst = edge_index[1]
    pad = jnp.full((E_PAD - E,), N, dtype=jnp.int32)

    def perm(a):
        return 4 * (a % NP4) + a // NP4

    srcf = perm(jnp.concatenate([src, pad])).reshape(TOT, K)
    dstf = perm(jnp.concatenate([dst, pad])).reshape(TOT, K)
    dst3 = dstf.reshape(NW, C, K)
    srcn = srcf[:NS * CN].reshape(NS, CN, K)
    srcs = srcf[NS * CN:].reshape(NS, CS, K)
    dstn = dstf[:NS * CN].reshape(NS, CN, K)
    dsts = dstf[NS * CN:].reshape(NS, CS, K)

    x_pad = jnp.zeros((N_PAD, D), jnp.float32).at[:N].set(x)
    zeros1 = jnp.zeros((N_PAD,), jnp.float32)
    zeros2 = jnp.zeros((N_PAD, H), jnp.float32)
    ones = jnp.ones((K,), jnp.float32)

    eye4 = jnp.eye(4, dtype=jnp.float32)
    w2d = jnp.kron(eye4, W2)
    woutd = jnp.kron(eye4, W_out)
    b1p = jnp.tile(b1, 4).reshape(1, HP)
    b2p = jnp.tile(b2, 4).reshape(1, HP)
    boutp = jnp.tile(b_out, 4).reshape(1, 4)

    degp4 = _sc_degree(dst3, zeros1, ones).reshape(NC, NP4, 4)
    hs1 = _tc1(x_pad, W1, degp4)
    aggp1 = _sc_aggregate(hs1.reshape(N_PAD, H), srcn, dstn, srcs, dsts,
                          zeros2)
    r, hs2 = _tc2(aggp1.reshape(NC, NP4, HP), hs1, degp4, b1p, w2d)
    aggp2 = _sc_aggregate(hs2.reshape(N_PAD, H), srcn, dstn, srcs, dsts,
                          zeros2)
    out = _tc3(aggp2.reshape(NC, NP4, HP), hs2, r, degp4, b2p, woutd, boutp)
    return out.T.reshape(N_PAD, 1)[:N]
